# Initial kernel scaffold; baseline (speedup 1.0000x reference)
import jax
import jax.numpy as jnp
from jax.experimental import pallas as pl


def kernel(boxes_cache, proposal_boxes, proposal_logits):
    raise NotImplementedError("write your pallas kernel here")


# speedup vs baseline: 36.6958x; 36.6958x over previous
import jax
import jax.numpy as jnp
from jax.experimental import pallas as pl
from jax.experimental.pallas import tpu as pltpu

_IMG_W = 1333.0
_IMG_H = 800.0
_NMS_THR = 0.1
_SCORE_THR = 0.05
_NC = 10000
_NP = 5000
_N = 15000
_B = 128
_NPAD = 15360
_NB = _NPAD // _B


def _prep_body(x1, y1, x2, y2, raw, mc, mp, out):
    x1v, y1v, x2v, y2v = x1[...], y1[...], x2[...], y2[...]
    raws = raw[...]
    mcv, mpv = mc[...], mp[...]
    ltx = jnp.maximum(0.0, x1v)
    lty = jnp.maximum(0.0, y1v)
    rbx = jnp.minimum(_IMG_W, x2v)
    rby = jnp.minimum(_IMG_H, y2v)
    wx = jnp.maximum(rbx - ltx, 0.0)
    wy = jnp.maximum(rby - lty, 0.0)
    inter = wx * wy
    area = jnp.maximum(x2v - x1v, 0.0) * jnp.maximum(y2v - y1v, 0.0)
    iof = inter / jnp.maximum(area, 1e-12)
    iof = jnp.where(iof < 0.3, 0.0, iof)
    sc_c = iof * raws
    sc_p = 1.0 / (1.0 + jnp.exp(-raws))
    score = mcv * sc_c + mpv * sc_p
    cx1 = jnp.minimum(jnp.maximum(x1v, 0.0), _IMG_W)
    cy1 = jnp.minimum(jnp.maximum(y1v, 0.0), _IMG_H)
    cx2 = jnp.minimum(jnp.maximum(x2v, 0.0), _IMG_W)
    cy2 = jnp.minimum(jnp.maximum(y2v, 0.0), _IMG_H)
    iscache = mcv > 0.0
    out[0:1, :] = jnp.where(iscache, cx1, x1v)
    out[1:2, :] = jnp.where(iscache, cy1, y1v)
    out[2:3, :] = jnp.where(iscache, cx2, x2v)
    out[3:4, :] = jnp.where(iscache, cy2, y2v)
    out[4:5, :] = score
    valid = score > _SCORE_THR
    out[5:6, :] = valid.astype(jnp.float32)
    out[6:7, :] = -jnp.where(valid, score, -jnp.inf)
    out[7:8, :] = jnp.zeros_like(score)


def _nms_body(bsc, x1r, y1r, x2r, y2r, scr, valr, outr, keep_s, area_s):
    x1v, y1v, x2v, y2v = x1r[...], y1r[...], x2r[...], y2r[...]
    area_s[...] = jnp.maximum(x2v - x1v, 0.0) * jnp.maximum(y2v - y1v, 0.0)
    keep_s[...] = valr[...]

    def blk(b, _):
        base = b * _B
        rsl = pl.ds(base, _B)
        bb = bsc[rsl, :]
        rx1, ry1 = bb[:, 0:1], bb[:, 1:2]
        rx2, ry2 = bb[:, 2:3], bb[:, 3:4]
        ra = jnp.maximum(rx2 - rx1, 0.0) * jnp.maximum(ry2 - ry1, 0.0)

        def iou_tile(csl):
            cx1 = x1r[0:1, csl]
            cy1 = y1r[0:1, csl]
            cx2 = x2r[0:1, csl]
            cy2 = y2r[0:1, csl]
            ltx = jnp.maximum(rx1, cx1)
            lty = jnp.maximum(ry1, cy1)
            rbx = jnp.minimum(rx2, cx2)
            rby = jnp.minimum(ry2, cy2)
            wx = jnp.maximum(rbx - ltx, 0.0)
            wy = jnp.maximum(rby - lty, 0.0)
            inter = wx * wy
            ca = area_s[0:1, csl]
            union = jnp.maximum(ra + ca - inter, 1e-12)
            return inter / union

        iou = iou_tile(rsl)
        row_i = jax.lax.broadcasted_iota(jnp.int32, (_B, _B), 0)
        col_i = jax.lax.broadcasted_iota(jnp.int32, (_B, _B), 1)
        s_ut = jnp.where((iou > _NMS_THR) & (row_i < col_i), 1.0, 0.0)
        kb0 = keep_s[0:1, rsl]

        def w_cond(c):
            return c[1]

        def w_body(c):
            kb, _ = c
            supp = jax.lax.dot_general(
                kb, s_ut, (((1,), (0,)), ((), ())),
                preferred_element_type=jnp.float32)
            nkb = jnp.where(supp > 0.5, 0.0, kb0)
            return nkb, jnp.any(nkb != kb)

        kb, _ = jax.lax.while_loop(w_cond, w_body, (kb0, jnp.bool_(True)))
        keep_s[0:1, rsl] = kb

        @pl.when(jnp.sum(kb) > 0.0)
        def _():
            def ct_body(t, _):
                csl = pl.ds(t * _B, _B)
                kcol = keep_s[0:1, csl]

                @pl.when(jnp.sum(kcol) > 0.0)
                def _():
                    s_x = jnp.where(iou_tile(csl) > _NMS_THR, 1.0, 0.0)
                    supp = jax.lax.dot_general(
                        kb, s_x, (((1,), (0,)), ((), ())),
                        preferred_element_type=jnp.float32)
                    keep_s[0:1, csl] = jnp.where(supp > 0.5, 0.0, kcol)

                return 0

            jax.lax.fori_loop(b + 1, _NB, ct_body, 0)

        return 0

    jax.lax.fori_loop(0, _NB, blk, 0)

    k = keep_s[...]
    outr[0:1, :] = x1v * k
    outr[1:2, :] = y1v * k
    outr[2:3, :] = x2v * k
    outr[3:4, :] = y2v * k
    s = jnp.clip(scr[...], 1e-6, 1.0 - 1e-6)
    inv_sig = jnp.log(s) - jnp.log1p(-s)
    outr[4:5, :] = inv_sig * k
    outr[5:6, :] = k
    outr[6:8, :] = jnp.zeros((2, _NPAD), jnp.float32)


def kernel(boxes_cache, proposal_boxes, proposal_logits):
    boxes_cache = boxes_cache.astype(jnp.float32)
    proposal_boxes = proposal_boxes.astype(jnp.float32)
    proposal_logits = proposal_logits.astype(jnp.float32)
    npad = _NPAD - _N
    zpad = jnp.zeros((npad,), jnp.float32)

    def row(c, p):
        return jnp.concatenate([c, p, zpad])[None, :]

    x1 = row(boxes_cache[:, 0], proposal_boxes[:, 0])
    y1 = row(boxes_cache[:, 1], proposal_boxes[:, 1])
    x2 = row(boxes_cache[:, 2], proposal_boxes[:, 2])
    y2 = row(boxes_cache[:, 3], proposal_boxes[:, 3])
    raw = row(boxes_cache[:, 4], proposal_logits)
    mc = row(jnp.ones((_NC,), jnp.float32), jnp.zeros((_NP,), jnp.float32))
    mp = row(jnp.zeros((_NC,), jnp.float32), jnp.ones((_NP,), jnp.float32))

    prep = pl.pallas_call(
        _prep_body,
        out_shape=jax.ShapeDtypeStruct((8, _NPAD), jnp.float32),
    )(x1, y1, x2, y2, raw, mc, mp)

    order = jnp.argsort(prep[6])
    srt = prep[:, order]
    bsc = jnp.transpose(srt[0:4, :])

    outs = pl.pallas_call(
        _nms_body,
        out_shape=jax.ShapeDtypeStruct((8, _NPAD), jnp.float32),
        scratch_shapes=[
            pltpu.VMEM((1, _NPAD), jnp.float32),
            pltpu.VMEM((1, _NPAD), jnp.float32),
        ],
    )(bsc, srt[0:1], srt[1:2], srt[2:3], srt[3:4], srt[4:5], srt[5:6])

    inv_order = jnp.argsort(order)
    out_rows = outs[0:5, :]
    return jnp.transpose(out_rows[:, inv_order][:, :_N])


# speedup vs baseline: 116.0522x; 3.1625x over previous
import jax
import jax.numpy as jnp
from jax.experimental import pallas as pl
from jax.experimental.pallas import tpu as pltpu

_IMG_W = 1333.0
_IMG_H = 800.0
_NMS_THR = 0.1
_SCORE_THR = 0.05
_NC = 10000
_NP = 5000
_N = 15000
_B = 128
_NPAD = 15360
_NB = _NPAD // _B
_CT = 512
_NCT = _NPAD // _CT


def _prep_body(x1, y1, x2, y2, raw, mc, mp, out):
    x1v, y1v, x2v, y2v = x1[...], y1[...], x2[...], y2[...]
    raws = raw[...]
    mcv, mpv = mc[...], mp[...]
    ltx = jnp.maximum(0.0, x1v)
    lty = jnp.maximum(0.0, y1v)
    rbx = jnp.minimum(_IMG_W, x2v)
    rby = jnp.minimum(_IMG_H, y2v)
    wx = jnp.maximum(rbx - ltx, 0.0)
    wy = jnp.maximum(rby - lty, 0.0)
    inter = wx * wy
    area = jnp.maximum(x2v - x1v, 0.0) * jnp.maximum(y2v - y1v, 0.0)
    iof = inter / jnp.maximum(area, 1e-12)
    iof = jnp.where(iof < 0.3, 0.0, iof)
    sc_c = iof * raws
    sc_p = 1.0 / (1.0 + jnp.exp(-raws))
    score = mcv * sc_c + mpv * sc_p
    cx1 = jnp.minimum(jnp.maximum(x1v, 0.0), _IMG_W)
    cy1 = jnp.minimum(jnp.maximum(y1v, 0.0), _IMG_H)
    cx2 = jnp.minimum(jnp.maximum(x2v, 0.0), _IMG_W)
    cy2 = jnp.minimum(jnp.maximum(y2v, 0.0), _IMG_H)
    iscache = mcv > 0.0
    out[0:1, :] = jnp.where(iscache, cx1, x1v)
    out[1:2, :] = jnp.where(iscache, cy1, y1v)
    out[2:3, :] = jnp.where(iscache, cx2, x2v)
    out[3:4, :] = jnp.where(iscache, cy2, y2v)
    out[4:5, :] = score
    valid = score > _SCORE_THR
    out[5:6, :] = valid.astype(jnp.float32)
    out[6:7, :] = -jnp.where(valid, score, -jnp.inf)
    out[7:8, :] = jnp.zeros_like(score)


def _nms_body(bsc, x1r, y1r, x2r, y2r, scr, valr, outr, keep_s, area_s):
    x1v, y1v, x2v, y2v = x1r[...], y1r[...], x2r[...], y2r[...]
    area_s[...] = jnp.maximum(x2v - x1v, 0.0) * jnp.maximum(y2v - y1v, 0.0)
    keep_s[...] = valr[...]

    def blk(b, _):
        base = b * _B
        rsl = pl.ds(base, _B)
        bb = bsc[rsl, :]
        rx1, ry1 = bb[:, 0:1], bb[:, 1:2]
        rx2, ry2 = bb[:, 2:3], bb[:, 3:4]
        ra = jnp.maximum(rx2 - rx1, 0.0) * jnp.maximum(ry2 - ry1, 0.0)

        def iou_tile(csl):
            cx1 = x1r[0:1, csl]
            cy1 = y1r[0:1, csl]
            cx2 = x2r[0:1, csl]
            cy2 = y2r[0:1, csl]
            ltx = jnp.maximum(rx1, cx1)
            lty = jnp.maximum(ry1, cy1)
            rbx = jnp.minimum(rx2, cx2)
            rby = jnp.minimum(ry2, cy2)
            wx = jnp.maximum(rbx - ltx, 0.0)
            wy = jnp.maximum(rby - lty, 0.0)
            inter = wx * wy
            ca = area_s[0:1, csl]
            union = jnp.maximum(ra + ca - inter, 1e-12)
            return inter / union

        iou = iou_tile(rsl)
        row_i = jax.lax.broadcasted_iota(jnp.int32, (_B, _B), 0)
        col_i = jax.lax.broadcasted_iota(jnp.int32, (_B, _B), 1)
        s_ut = jnp.where((iou > _NMS_THR) & (row_i < col_i), 1.0, 0.0)
        kb0 = keep_s[0:1, rsl]

        def w_cond(c):
            return c[1]

        def w_body(c):
            kb, _ = c
            supp = jax.lax.dot_general(
                kb, s_ut, (((1,), (0,)), ((), ())),
                preferred_element_type=jnp.float32)
            nkb = jnp.where(supp > 0.5, 0.0, kb0)
            return nkb, jnp.any(nkb != kb)

        kb, _ = jax.lax.while_loop(w_cond, w_body, (kb0, jnp.bool_(True)))
        keep_s[0:1, rsl] = kb

        @pl.when(jnp.sum(kb) > 0.0)
        def _():
            def ct_body(t, _):
                cbase = t * _CT
                csl = pl.ds(cbase, _CT)
                kcol = keep_s[0:1, csl]
                s_x = jnp.where(iou_tile(csl) > _NMS_THR, 1.0, 0.0)
                supp = jax.lax.dot_general(
                    kb, s_x, (((1,), (0,)), ((), ())),
                    preferred_element_type=jnp.float32)
                colid = cbase + jax.lax.broadcasted_iota(jnp.int32, (1, _CT), 1)
                later = colid >= base + _B
                keep_s[0:1, csl] = jnp.where((supp > 0.5) & later, 0.0, kcol)
                return 0

            jax.lax.fori_loop((base + _B) // _CT, _NCT, ct_body, 0)

        return 0

    jax.lax.fori_loop(0, _NB, blk, 0)

    k = keep_s[...]
    outr[0:1, :] = x1v * k
    outr[1:2, :] = y1v * k
    outr[2:3, :] = x2v * k
    outr[3:4, :] = y2v * k
    s = jnp.clip(scr[...], 1e-6, 1.0 - 1e-6)
    inv_sig = jnp.log(s) - jnp.log1p(-s)
    outr[4:5, :] = inv_sig * k
    outr[5:6, :] = k
    outr[6:8, :] = jnp.zeros((2, _NPAD), jnp.float32)


def kernel(boxes_cache, proposal_boxes, proposal_logits):
    boxes_cache = boxes_cache.astype(jnp.float32)
    proposal_boxes = proposal_boxes.astype(jnp.float32)
    proposal_logits = proposal_logits.astype(jnp.float32)
    npad = _NPAD - _N
    zpad = jnp.zeros((npad,), jnp.float32)

    def row(c, p):
        return jnp.concatenate([c, p, zpad])[None, :]

    x1 = row(boxes_cache[:, 0], proposal_boxes[:, 0])
    y1 = row(boxes_cache[:, 1], proposal_boxes[:, 1])
    x2 = row(boxes_cache[:, 2], proposal_boxes[:, 2])
    y2 = row(boxes_cache[:, 3], proposal_boxes[:, 3])
    raw = row(boxes_cache[:, 4], proposal_logits)
    mc = row(jnp.ones((_NC,), jnp.float32), jnp.zeros((_NP,), jnp.float32))
    mp = row(jnp.zeros((_NC,), jnp.float32), jnp.ones((_NP,), jnp.float32))

    prep = pl.pallas_call(
        _prep_body,
        out_shape=jax.ShapeDtypeStruct((8, _NPAD), jnp.float32),
    )(x1, y1, x2, y2, raw, mc, mp)

    order = jnp.argsort(prep[6])
    srt = prep[:, order]
    bsc = jnp.transpose(srt[0:4, :])

    outs = pl.pallas_call(
        _nms_body,
        out_shape=jax.ShapeDtypeStruct((8, _NPAD), jnp.float32),
        scratch_shapes=[
            pltpu.VMEM((1, _NPAD), jnp.float32),
            pltpu.VMEM((1, _NPAD), jnp.float32),
        ],
    )(bsc, srt[0:1], srt[1:2], srt[2:3], srt[3:4], srt[4:5], srt[5:6])

    inv_order = jnp.argsort(order)
    out_rows = outs[0:5, :]
    return jnp.transpose(out_rows[:, inv_order][:, :_N])


# speedup vs baseline: 136.6469x; 1.1775x over previous
import jax
import jax.numpy as jnp
from jax.experimental import pallas as pl
from jax.experimental.pallas import tpu as pltpu

_IMG_W = 1333.0
_IMG_H = 800.0
_NMS_THR = 0.1
_SCORE_THR = 0.05
_NC = 10000
_NP = 5000
_N = 15000
_B = 128
_NPAD = 15360
_NB = _NPAD // _B
_CT = 512
_NCT = _NPAD // _CT


def _prep_body(x1, y1, x2, y2, raw, mc, mp, out):
    x1v, y1v, x2v, y2v = x1[...], y1[...], x2[...], y2[...]
    raws = raw[...]
    mcv, mpv = mc[...], mp[...]
    ltx = jnp.maximum(0.0, x1v)
    lty = jnp.maximum(0.0, y1v)
    rbx = jnp.minimum(_IMG_W, x2v)
    rby = jnp.minimum(_IMG_H, y2v)
    wx = jnp.maximum(rbx - ltx, 0.0)
    wy = jnp.maximum(rby - lty, 0.0)
    inter = wx * wy
    area = jnp.maximum(x2v - x1v, 0.0) * jnp.maximum(y2v - y1v, 0.0)
    iof = inter / jnp.maximum(area, 1e-12)
    iof = jnp.where(iof < 0.3, 0.0, iof)
    sc_c = iof * raws
    sc_p = 1.0 / (1.0 + jnp.exp(-raws))
    score = mcv * sc_c + mpv * sc_p
    cx1 = jnp.minimum(jnp.maximum(x1v, 0.0), _IMG_W)
    cy1 = jnp.minimum(jnp.maximum(y1v, 0.0), _IMG_H)
    cx2 = jnp.minimum(jnp.maximum(x2v, 0.0), _IMG_W)
    cy2 = jnp.minimum(jnp.maximum(y2v, 0.0), _IMG_H)
    iscache = mcv > 0.0
    out[0:1, :] = jnp.where(iscache, cx1, x1v)
    out[1:2, :] = jnp.where(iscache, cy1, y1v)
    out[2:3, :] = jnp.where(iscache, cx2, x2v)
    out[3:4, :] = jnp.where(iscache, cy2, y2v)
    out[4:5, :] = score
    valid = score > _SCORE_THR
    out[5:6, :] = valid.astype(jnp.float32)
    out[6:7, :] = -jnp.where(valid, score, -jnp.inf)
    out[7:8, :] = jnp.zeros_like(score)


def _nms_body(bsc, x1r, y1r, x2r, y2r, scr, valr, outr, keep_s, area_s):
    x1v, y1v, x2v, y2v = x1r[...], y1r[...], x2r[...], y2r[...]
    area_s[...] = jnp.maximum(x2v - x1v, 0.0) * jnp.maximum(y2v - y1v, 0.0)
    keep_s[...] = valr[...]

    def blk(b, _):
        base = b * _B
        rsl = pl.ds(base, _B)
        bb = bsc[rsl, :]
        rx1, ry1 = bb[:, 0:1], bb[:, 1:2]
        rx2, ry2 = bb[:, 2:3], bb[:, 3:4]
        ra = jnp.maximum(rx2 - rx1, 0.0) * jnp.maximum(ry2 - ry1, 0.0)

        def iou_tile(csl):
            cx1 = x1r[0:1, csl]
            cy1 = y1r[0:1, csl]
            cx2 = x2r[0:1, csl]
            cy2 = y2r[0:1, csl]
            ltx = jnp.maximum(rx1, cx1)
            lty = jnp.maximum(ry1, cy1)
            rbx = jnp.minimum(rx2, cx2)
            rby = jnp.minimum(ry2, cy2)
            wx = jnp.maximum(rbx - ltx, 0.0)
            wy = jnp.maximum(rby - lty, 0.0)
            inter = wx * wy
            ca = area_s[0:1, csl]
            union = jnp.maximum(ra + ca - inter, 1e-12)
            return inter / union

        iou = iou_tile(rsl)
        row_i = jax.lax.broadcasted_iota(jnp.int32, (_B, _B), 0)
        col_i = jax.lax.broadcasted_iota(jnp.int32, (_B, _B), 1)
        s_ut = jnp.where((iou > _NMS_THR) & (row_i < col_i), 1.0, 0.0)
        kb0 = keep_s[0:1, rsl]

        def w_cond(c):
            return c[1]

        def w_body(c):
            kb, _ = c
            supp = jax.lax.dot_general(
                kb, s_ut, (((1,), (0,)), ((), ())),
                preferred_element_type=jnp.float32)
            nkb = jnp.where(supp > 0.5, 0.0, kb0)
            return nkb, jnp.any(nkb != kb)

        kb, _ = jax.lax.while_loop(w_cond, w_body, (kb0, jnp.bool_(True)))
        keep_s[0:1, rsl] = kb
        eye = jnp.where(row_i == col_i, 1.0, 0.0)
        kb_col = jax.lax.dot_general(
            eye, kb, (((1,), (1,)), ((), ())),
            preferred_element_type=jnp.float32)

        @pl.when(jnp.sum(kb) > 0.0)
        def _():
            def ct_body(t, _):
                cbase = t * _CT
                csl = pl.ds(cbase, _CT)
                kcol = keep_s[0:1, csl]
                s_x = jnp.where(
                    (iou_tile(csl) > _NMS_THR) & (kb_col > 0.5), 1.0, 0.0)
                supp = jnp.max(s_x, axis=0, keepdims=True)
                colid = cbase + jax.lax.broadcasted_iota(jnp.int32, (1, _CT), 1)
                later = colid >= base + _B
                keep_s[0:1, csl] = jnp.where((supp > 0.5) & later, 0.0, kcol)
                return 0

            jax.lax.fori_loop((base + _B) // _CT, _NCT, ct_body, 0)

        return 0

    jax.lax.fori_loop(0, _NB, blk, 0)

    k = keep_s[...]
    outr[0:1, :] = x1v * k
    outr[1:2, :] = y1v * k
    outr[2:3, :] = x2v * k
    outr[3:4, :] = y2v * k
    s = jnp.clip(scr[...], 1e-6, 1.0 - 1e-6)
    inv_sig = jnp.log(s) - jnp.log1p(-s)
    outr[4:5, :] = inv_sig * k
    outr[5:6, :] = k
    outr[6:8, :] = jnp.zeros((2, _NPAD), jnp.float32)


def kernel(boxes_cache, proposal_boxes, proposal_logits):
    boxes_cache = boxes_cache.astype(jnp.float32)
    proposal_boxes = proposal_boxes.astype(jnp.float32)
    proposal_logits = proposal_logits.astype(jnp.float32)
    npad = _NPAD - _N
    zpad = jnp.zeros((npad,), jnp.float32)

    def row(c, p):
        return jnp.concatenate([c, p, zpad])[None, :]

    x1 = row(boxes_cache[:, 0], proposal_boxes[:, 0])
    y1 = row(boxes_cache[:, 1], proposal_boxes[:, 1])
    x2 = row(boxes_cache[:, 2], proposal_boxes[:, 2])
    y2 = row(boxes_cache[:, 3], proposal_boxes[:, 3])
    raw = row(boxes_cache[:, 4], proposal_logits)
    mc = row(jnp.ones((_NC,), jnp.float32), jnp.zeros((_NP,), jnp.float32))
    mp = row(jnp.zeros((_NC,), jnp.float32), jnp.ones((_NP,), jnp.float32))

    prep = pl.pallas_call(
        _prep_body,
        out_shape=jax.ShapeDtypeStruct((8, _NPAD), jnp.float32),
    )(x1, y1, x2, y2, raw, mc, mp)

    order = jnp.argsort(prep[6])
    srt = prep[:, order]
    bsc = jnp.transpose(srt[0:4, :])

    outs = pl.pallas_call(
        _nms_body,
        out_shape=jax.ShapeDtypeStruct((8, _NPAD), jnp.float32),
        scratch_shapes=[
            pltpu.VMEM((1, _NPAD), jnp.float32),
            pltpu.VMEM((1, _NPAD), jnp.float32),
        ],
    )(bsc, srt[0:1], srt[1:2], srt[2:3], srt[3:4], srt[4:5], srt[5:6])

    inv_order = jnp.argsort(order)
    out_rows = outs[0:5, :]
    return jnp.transpose(out_rows[:, inv_order][:, :_N])
